# baseline (speedup 1.0000x reference)
import functools

import jax
import jax.numpy as jnp
from jax import lax
from jax.experimental import pallas as pl
from jax.experimental.pallas import tpu as pltpu
from jax.experimental.pallas import tpu_sc as plsc

_C = 1.0
_NC = 2
_NS = 16
_L = 16
_NW = _NC * _NS
_BLK = 80


def _node_table_body(x_ref, wp_ref, b_ref, tab_ref):
    x = x_ref[...]
    x0 = x[:, 0:1]
    total = jnp.sum(x * x, axis=1, keepdims=True)
    nsq = jnp.maximum(total - x0 * x0, 0.0)
    norm = jnp.maximum(jnp.sqrt(nsq), 1e-8)
    x0c = jnp.maximum(x0, 1.0 + 1e-6)
    dist = jnp.log(x0c + jnp.sqrt((x0c - 1.0) * (x0c + 1.0)))
    s = dist / norm
    mu = jnp.dot(x, wp_ref[...], preferred_element_type=jnp.float32) * s
    mu = mu + b_ref[0:1, :]
    mn = jnp.maximum(jnp.sqrt(jnp.sum(mu * mu, axis=1, keepdims=True)), 1e-8)
    e = jnp.exp(mn)
    ei = 1.0 / e
    ch = 0.5 * (e + ei)
    sh = 0.5 * (e - ei)
    tail = (sh / mn) * mu
    used = tail[:, : x.shape[1] - 1]
    tab_ref[...] = jnp.concatenate([ch, used], axis=1)


def _make_node_table(x, wp, b8, bn):
    n, d_in = x.shape
    grid = n // bn
    return pl.pallas_call(
        _node_table_body,
        grid=(grid,),
        in_specs=[
            pl.BlockSpec((bn, d_in), lambda i: (i, 0)),
            pl.BlockSpec((d_in, d_in), lambda i: (0, 0)),
            pl.BlockSpec((8, d_in), lambda i: (0, 0)),
        ],
        out_specs=pl.BlockSpec((bn, d_in), lambda i: (i, 0)),
        out_shape=jax.ShapeDtypeStruct((n, d_in), jnp.float32),
    )(x, wp, b8)


_CHUNK = 5


def _make_edge_kernel(e_total, n_rows, d_in):
    mesh = plsc.VectorSubcoreMesh(
        core_axis_name="c", subcore_axis_name="s", num_cores=_NC
    )
    ept = e_total // _NW
    nblk = ept // _BLK
    nch = nblk // _CHUNK
    groups = _BLK // _L
    rows_per_blk = 2 * _BLK
    iwords = _CHUNK * rows_per_blk
    cwords = _CHUNK * _BLK
    stage_rows = n_rows // _NS

    @functools.partial(
        pl.kernel,
        mesh=mesh,
        compiler_params=pltpu.CompilerParams(needs_layout_passes=False),
        out_type=jax.ShapeDtypeStruct((e_total,), jnp.float32),
        scratch_types=[
            pltpu.VMEM_SHARED((n_rows, d_in), jnp.float32),
            pltpu.VMEM((iwords,), jnp.int32),
            pltpu.VMEM((iwords,), jnp.int32),
            pltpu.VMEM((rows_per_blk, d_in), jnp.float32),
            pltpu.VMEM((rows_per_blk, d_in), jnp.float32),
            pltpu.VMEM((cwords,), jnp.float32),
            pltpu.SemaphoreType.DMA,
            pltpu.SemaphoreType.DMA,
            pltpu.SemaphoreType.DMA,
        ],
    )
    def edge_kernel(tab_hbm, idx_hbm, out_hbm,
                    stab, ibuf0, ibuf1, rows0, rows1, res_v,
                    sem_r0, sem_r1, sem_i):
        cid = lax.axis_index("c")
        sid = lax.axis_index("s")
        wid = sid * _NC + cid
        iota = lax.iota(jnp.int32, _L)
        tile_idx_base = wid * 2 * ept
        tile_out_base = wid * ept

        pltpu.sync_copy(tab_hbm.at[pl.ds(sid * stage_rows, stage_rows)],
                        stab.at[pl.ds(sid * stage_rows, stage_rows)])
        plsc.subcore_barrier()

        def idx_src(c):
            return idx_hbm.at[pl.ds(tile_idx_base + c * iwords, iwords)]

        def gather_desc(ibuf, j, rbuf, sem):
            idx = ibuf.at[pl.ds(j * rows_per_blk, rows_per_blk)]
            return pltpu.make_async_copy(stab.at[idx], rbuf, sem)

        rbufs = (rows0, rows1)
        rsems = (sem_r0, sem_r1)

        def compute_block(buf_cur, j):
            def dim_body(d, accs):
                col = jnp.full((_L,), d, jnp.int32)
                out = []
                for g in range(groups):
                    rs = iota + (g * _L)
                    rd = rs + _BLK
                    a = plsc.load_gather(buf_cur, [rs, col])
                    b2 = plsc.load_gather(buf_cur, [rd, col])
                    out.append(accs[g] + a * b2)
                return tuple(out)

            accs = lax.fori_loop(
                0, d_in, dim_body,
                tuple(jnp.zeros((_L,), jnp.float32) for _ in range(groups)),
                unroll=4)
            col0 = jnp.zeros((_L,), jnp.int32)
            for g in range(groups):
                rs = iota + (g * _L)
                hs = plsc.load_gather(buf_cur, [rs, col0])
                hd = plsc.load_gather(buf_cur, [rs + _BLK, col0])
                l_inner = accs[g] - 2.0 * hs * hd
                t = -_C - l_inner
                r = jnp.minimum(jnp.maximum(t, 1e-10), 1.0)
                res_v[pl.ds(j * _BLK + g * _L, _L)] = jnp.exp(-r)

        def process_chunk(c, icur, inext, par):
            for j in range(_CHUNK):
                p = (par + j) % 2
                q = 1 - p
                gather_desc(icur, j, rbufs[p], rsems[p]).wait()
                if j < _CHUNK - 1:
                    gather_desc(icur, j + 1, rbufs[q], rsems[q]).start()
                else:
                    @pl.when(c < nch - 1)
                    def _():
                        pltpu.make_async_copy(idx_src(c + 1), inext,
                                              sem_i).wait()
                        gather_desc(inext, 0, rbufs[q], rsems[q]).start()
                compute_block(rbufs[p], j)
            pltpu.sync_copy(res_v,
                            out_hbm.at[pl.ds(tile_out_base + c * cwords,
                                             cwords)])
            @pl.when(c < nch - 2)
            def _():
                pltpu.async_copy(idx_src(c + 2), icur, sem_i)

        pltpu.sync_copy(idx_src(0), ibuf0)
        gather_desc(ibuf0, 0, rows0, sem_r0).start()

        @pl.when(nch > 1)
        def _():
            pltpu.async_copy(idx_src(1), ibuf1, sem_i)

        def chunk_body(c, carry):
            @pl.when(lax.rem(c, 2) == 0)
            def _even():
                process_chunk(c, ibuf0, ibuf1, 0)

            @pl.when(lax.rem(c, 2) == 1)
            def _odd():
                process_chunk(c, ibuf1, ibuf0, 1)

            return carry

        lax.fori_loop(0, nch, chunk_body, 0)

    return edge_kernel


def kernel(x, edge_index, W, b):
    n, d_in = x.shape
    e = edge_index.shape[1]
    x = x.astype(jnp.float32)
    wp = jnp.concatenate(
        [jnp.zeros((1, d_in), jnp.float32), W.astype(jnp.float32)], axis=0
    )
    b8 = jnp.broadcast_to(b.astype(jnp.float32), (8, d_in))

    bn = 256
    n_pad = ((n + bn - 1) // bn) * bn
    xp = x if n_pad == n else jnp.pad(x, ((0, n_pad - n), (0, 0)))
    tab = _make_node_table(xp, wp, b8, bn)

    src = edge_index[0].astype(jnp.int32)
    dst = edge_index[1].astype(jnp.int32)
    chunk = _NW * _BLK
    e_pad = ((e + chunk - 1) // chunk) * chunk
    if e_pad != e:
        src = jnp.pad(src, (0, e_pad - e))
        dst = jnp.pad(dst, (0, e_pad - e))
    nblk = e_pad // chunk
    s3 = src.reshape(_NW, nblk, _BLK)
    d3 = dst.reshape(_NW, nblk, _BLK)
    idx_cat = jnp.concatenate([s3, d3], axis=2).reshape(2 * e_pad)

    res = _make_edge_kernel(e_pad, n_pad, d_in)(tab, idx_cat)
    if e_pad != e:
        res = res[:e]
    return (edge_index, res, (n, n))


# speedup vs baseline: 2.4366x; 2.4366x over previous
import functools

import jax
import jax.numpy as jnp
from jax import lax
from jax.experimental import pallas as pl
from jax.experimental.pallas import tpu as pltpu
from jax.experimental.pallas import tpu_sc as plsc

_C = 1.0
_NC = 2
_NS = 16
_L = 16
_SW = 8
_CE = 4096
_STRIP = _CE // _NS


def _node_table_body(x_ref, wp_ref, b_ref, tab_ref):
    x = x_ref[...]
    x0 = x[:, 0:1]
    total = jnp.sum(x * x, axis=1, keepdims=True)
    nsq = jnp.maximum(total - x0 * x0, 0.0)
    norm = jnp.maximum(jnp.sqrt(nsq), 1e-8)
    x0c = jnp.maximum(x0, 1.0 + 1e-6)
    dist = jnp.log(x0c + jnp.sqrt((x0c - 1.0) * (x0c + 1.0)))
    s = dist / norm
    mu = jnp.dot(x, wp_ref[...], preferred_element_type=jnp.float32) * s
    mu = mu + b_ref[0:1, :]
    mn = jnp.maximum(jnp.sqrt(jnp.sum(mu * mu, axis=1, keepdims=True)), 1e-8)
    e = jnp.exp(mn)
    ei = 1.0 / e
    ch = 0.5 * (e + ei)
    sh = 0.5 * (e - ei)
    tail = (sh / mn) * mu
    used = tail[:, : x.shape[1] - 1]
    tab_ref[...] = jnp.concatenate([ch, used], axis=1)


def _make_node_table(x, wp, b8, bn):
    n, d_in = x.shape
    grid = n // bn
    return pl.pallas_call(
        _node_table_body,
        grid=(grid,),
        in_specs=[
            pl.BlockSpec((bn, d_in), lambda i: (i, 0)),
            pl.BlockSpec((d_in, d_in), lambda i: (0, 0)),
            pl.BlockSpec((8, d_in), lambda i: (0, 0)),
        ],
        out_specs=pl.BlockSpec((bn, d_in), lambda i: (i, 0)),
        out_shape=jax.ShapeDtypeStruct((n, d_in), jnp.float32),
    )(x, wp, b8)


def _make_edge_kernel(e_total, n_rows):
    mesh = plsc.VectorSubcoreMesh(
        core_axis_name="c", subcore_axis_name="s", num_cores=_NC
    )
    epc = e_total // _NC
    nch = epc // _CE
    groups = _CE // _L
    slab_words = n_rows * _SW

    @functools.partial(
        pl.kernel,
        mesh=mesh,
        compiler_params=pltpu.CompilerParams(needs_layout_passes=False),
        out_type=jax.ShapeDtypeStruct((e_total,), jnp.float32),
        scratch_types=[
            pltpu.VMEM((slab_words,), jnp.float32),
            pltpu.VMEM((2 * _CE,), jnp.int32),
            pltpu.VMEM((2 * _CE,), jnp.int32),
            pltpu.VMEM((_CE,), jnp.float32),
            pltpu.VMEM((_NS, _STRIP), jnp.float32),
            pltpu.VMEM((_STRIP,), jnp.float32),
            pltpu.VMEM_SHARED((_NS, _CE), jnp.float32),
            pltpu.VMEM_SHARED((_NS, _CE), jnp.float32),
            pltpu.SemaphoreType.DMA,
            pltpu.SemaphoreType.DMA,
        ],
    )
    def edge_kernel(tabt_hbm, idx_hbm, out_hbm,
                    slab_v, idx0, idx1, part_v, sum_v, res_v,
                    xch0, xch1, sem_i0, sem_i1):
        cid = lax.axis_index("c")
        sid = lax.axis_index("s")
        iota = lax.iota(jnp.int32, _L)
        sgn0 = jnp.where(sid == 0, -1.0, 1.0)
        sgn = jnp.zeros((_L,), jnp.float32) + sgn0

        pltpu.sync_copy(tabt_hbm.at[sid], slab_v)

        idx_base = cid * nch * 2 * _CE
        out_base = cid * nch * _CE

        def idx_src(c):
            return idx_hbm.at[pl.ds(idx_base + c * 2 * _CE, 2 * _CE)]

        ibufs = (idx0, idx1)
        isems = (sem_i0, sem_i1)
        xbufs = (xch0, xch1)

        pltpu.sync_copy(idx_src(0), idx0)

        @pl.when(nch > 1)
        def _():
            pltpu.async_copy(idx_src(1), idx1, sem_i1)

        def process_chunk(c, ibuf, isem, ibuf_pf, isem_pf, xch):
            @pl.when(c > 0)
            def _():
                pltpu.make_async_copy(idx_src(c), ibuf, isem).wait()

            def group_body(g, carry):
                goff = g * _L
                sidx = ibuf[pl.ds(goff, _L)]
                didx = ibuf[pl.ds(_CE + goff, _L)]
                rs = sidx * _SW
                rd = didx * _SW
                a0 = plsc.load_gather(slab_v, [rs])
                b0 = plsc.load_gather(slab_v, [rd])
                acc = (a0 * sgn) * b0
                for d in range(1, _SW):
                    a = plsc.load_gather(slab_v, [rs + d])
                    b2 = plsc.load_gather(slab_v, [rd + d])
                    acc = acc + a * b2
                part_v[pl.ds(goff, _L)] = acc
                return carry

            lax.fori_loop(0, groups, group_body, 0, unroll=2)

            pltpu.sync_copy(part_v, xch.at[sid])
            plsc.subcore_barrier()
            pltpu.sync_copy(xch.at[:, pl.ds(sid * _STRIP, _STRIP)], sum_v)

            for v in range(_STRIP // _L):
                tot = sum_v[0, pl.ds(v * _L, _L)]
                for r in range(1, _NS):
                    tot = tot + sum_v[r, pl.ds(v * _L, _L)]
                t = -_C - tot
                rr = jnp.minimum(jnp.maximum(t, 1e-10), 1.0)
                res_v[pl.ds(v * _L, _L)] = jnp.exp(-rr)
            pltpu.sync_copy(
                res_v,
                out_hbm.at[pl.ds(out_base + c * _CE + sid * _STRIP, _STRIP)])

            @pl.when(c < nch - 2)
            def _():
                pltpu.async_copy(idx_src(c + 2), ibuf, isem)

        def chunk_body(c, carry):
            @pl.when(lax.rem(c, 2) == 0)
            def _even():
                process_chunk(c, idx0, sem_i0, idx1, sem_i1, xch0)

            @pl.when(lax.rem(c, 2) == 1)
            def _odd():
                process_chunk(c, idx1, sem_i1, idx0, sem_i0, xch1)

            return carry

        lax.fori_loop(0, nch, chunk_body, 0)

    return edge_kernel


def kernel(x, edge_index, W, b):
    n, d_in = x.shape
    e = edge_index.shape[1]
    x = x.astype(jnp.float32)
    wp = jnp.concatenate(
        [jnp.zeros((1, d_in), jnp.float32), W.astype(jnp.float32)], axis=0
    )
    b8 = jnp.broadcast_to(b.astype(jnp.float32), (8, d_in))

    bn = 256
    n_pad = ((n + bn - 1) // bn) * bn
    xp = x if n_pad == n else jnp.pad(x, ((0, n_pad - n), (0, 0)))
    tab = _make_node_table(xp, wp, b8, bn)
    tabt = tab.reshape(n_pad, _NS, _SW).transpose(1, 0, 2).reshape(
        _NS, n_pad * _SW)

    src = edge_index[0].astype(jnp.int32)
    dst = edge_index[1].astype(jnp.int32)
    chunk = _NC * _CE
    e_pad = ((e + chunk - 1) // chunk) * chunk
    if e_pad != e:
        src = jnp.pad(src, (0, e_pad - e))
        dst = jnp.pad(dst, (0, e_pad - e))
    nch = e_pad // chunk
    s3 = src.reshape(_NC, nch, _CE)
    d3 = dst.reshape(_NC, nch, _CE)
    idx_cat = jnp.concatenate([s3, d3], axis=2).reshape(2 * e_pad)

    res = _make_edge_kernel(e_pad, n_pad)(tabt, idx_cat)
    if e_pad != e:
        res = res[:e]
    return (edge_index, res, (n, n))


# speedup vs baseline: 2.9449x; 1.2086x over previous
import functools

import jax
import jax.numpy as jnp
from jax import lax
from jax.experimental import pallas as pl
from jax.experimental.pallas import tpu as pltpu
from jax.experimental.pallas import tpu_sc as plsc

_C = 1.0
_NC = 2
_NS = 16
_L = 16
_SW = 8
_CE = 4096
_STRIP = _CE // _NS


def _node_table_body(x_ref, wp_ref, b_ref, tab_ref):
    x = x_ref[...]
    x0 = x[:, 0:1]
    total = jnp.sum(x * x, axis=1, keepdims=True)
    nsq = jnp.maximum(total - x0 * x0, 0.0)
    norm = jnp.maximum(jnp.sqrt(nsq), 1e-8)
    x0c = jnp.maximum(x0, 1.0 + 1e-6)
    dist = jnp.log(x0c + jnp.sqrt((x0c - 1.0) * (x0c + 1.0)))
    s = dist / norm
    mu = jnp.dot(x, wp_ref[...], preferred_element_type=jnp.float32) * s
    mu = mu + b_ref[0:1, :]
    mn = jnp.maximum(jnp.sqrt(jnp.sum(mu * mu, axis=1, keepdims=True)), 1e-8)
    e = jnp.exp(mn)
    ei = 1.0 / e
    ch = 0.5 * (e + ei)
    sh = 0.5 * (e - ei)
    tail = (sh / mn) * mu
    used = tail[:, : x.shape[1] - 1]
    tab_ref[...] = jnp.concatenate([ch, used], axis=1)


def _make_node_table(x, wp, b8, bn):
    n, d_in = x.shape
    grid = n // bn
    return pl.pallas_call(
        _node_table_body,
        grid=(grid,),
        in_specs=[
            pl.BlockSpec((bn, d_in), lambda i: (i, 0)),
            pl.BlockSpec((d_in, d_in), lambda i: (0, 0)),
            pl.BlockSpec((8, d_in), lambda i: (0, 0)),
        ],
        out_specs=pl.BlockSpec((bn, d_in), lambda i: (i, 0)),
        out_shape=jax.ShapeDtypeStruct((n, d_in), jnp.float32),
    )(x, wp, b8)


def _make_edge_kernel(e_total, n_rows):
    mesh = plsc.VectorSubcoreMesh(
        core_axis_name="c", subcore_axis_name="s", num_cores=_NC
    )
    epc = e_total // _NC
    nch = epc // _CE
    groups = _CE // _L
    slab_words = n_rows * _SW

    @functools.partial(
        pl.kernel,
        mesh=mesh,
        compiler_params=pltpu.CompilerParams(needs_layout_passes=False),
        out_type=jax.ShapeDtypeStruct((e_total,), jnp.float32),
        scratch_types=[
            pltpu.VMEM((slab_words,), jnp.float32),
            pltpu.VMEM((2 * _CE,), jnp.int32),
            pltpu.VMEM((2 * _CE,), jnp.int32),
            pltpu.VMEM((_CE,), jnp.float32),
            pltpu.VMEM((_NS, _STRIP), jnp.float32),
            pltpu.VMEM((_STRIP,), jnp.float32),
            pltpu.VMEM_SHARED((_NS, _CE), jnp.float32),
            pltpu.VMEM_SHARED((_NS, _CE), jnp.float32),
            pltpu.SemaphoreType.DMA,
            pltpu.SemaphoreType.DMA,
        ],
    )
    def edge_kernel(tabt_hbm, idx_hbm, out_hbm,
                    slab_v, idx0, idx1, part_v, sum_v, res_v,
                    xch0, xch1, sem_i0, sem_i1):
        cid = lax.axis_index("c")
        sid = lax.axis_index("s")
        iota = lax.iota(jnp.int32, _L)
        sgn0 = jnp.where(sid == 0, -1.0, 1.0)
        sgn = jnp.zeros((_L,), jnp.float32) + sgn0

        pltpu.sync_copy(tabt_hbm.at[sid], slab_v)

        idx_base = cid * nch * 2 * _CE
        out_base = cid * nch * _CE

        def idx_src(c):
            return idx_hbm.at[pl.ds(idx_base + c * 2 * _CE, 2 * _CE)]

        ibufs = (idx0, idx1)
        isems = (sem_i0, sem_i1)
        xbufs = (xch0, xch1)

        pltpu.sync_copy(idx_src(0), idx0)

        @pl.when(nch > 1)
        def _():
            pltpu.async_copy(idx_src(1), idx1, sem_i1)

        def process_chunk(c, ibuf, isem, ibuf_pf, isem_pf, xch):
            @pl.when(c > 0)
            def _():
                pltpu.make_async_copy(idx_src(c), ibuf, isem).wait()

            def group_body(g, carry):
                goff = g * _L
                sidx = ibuf[pl.ds(goff, _L)]
                didx = ibuf[pl.ds(_CE + goff, _L)]
                rs = sidx * _SW
                rd = didx * _SW
                av = [plsc.load_gather(slab_v, [rs + d] if d else [rs])
                      for d in range(_SW)]
                bv = [plsc.load_gather(slab_v, [rd + d] if d else [rd])
                      for d in range(_SW)]
                prods = [(av[0] * sgn) * bv[0]]
                prods += [av[d] * bv[d] for d in range(1, _SW)]
                while len(prods) > 1:
                    prods = [prods[i] + prods[i + 1]
                             for i in range(0, len(prods) - 1, 2)] + (
                                 [prods[-1]] if len(prods) % 2 else [])
                part_v[pl.ds(goff, _L)] = prods[0]
                return carry

            lax.fori_loop(0, groups, group_body, 0, unroll=4)

            pltpu.sync_copy(part_v, xch.at[sid])
            plsc.subcore_barrier()
            pltpu.sync_copy(xch.at[:, pl.ds(sid * _STRIP, _STRIP)], sum_v)

            for v in range(_STRIP // _L):
                tot = sum_v[0, pl.ds(v * _L, _L)]
                for r in range(1, _NS):
                    tot = tot + sum_v[r, pl.ds(v * _L, _L)]
                t = -_C - tot
                rr = jnp.minimum(jnp.maximum(t, 1e-10), 1.0)
                res_v[pl.ds(v * _L, _L)] = jnp.exp(-rr)
            pltpu.sync_copy(
                res_v,
                out_hbm.at[pl.ds(out_base + c * _CE + sid * _STRIP, _STRIP)])

            @pl.when(c < nch - 2)
            def _():
                pltpu.async_copy(idx_src(c + 2), ibuf, isem)

        def chunk_body(c, carry):
            @pl.when(lax.rem(c, 2) == 0)
            def _even():
                process_chunk(c, idx0, sem_i0, idx1, sem_i1, xch0)

            @pl.when(lax.rem(c, 2) == 1)
            def _odd():
                process_chunk(c, idx1, sem_i1, idx0, sem_i0, xch1)

            return carry

        lax.fori_loop(0, nch, chunk_body, 0)

    return edge_kernel


def kernel(x, edge_index, W, b):
    n, d_in = x.shape
    e = edge_index.shape[1]
    x = x.astype(jnp.float32)
    wp = jnp.concatenate(
        [jnp.zeros((1, d_in), jnp.float32), W.astype(jnp.float32)], axis=0
    )
    b8 = jnp.broadcast_to(b.astype(jnp.float32), (8, d_in))

    bn = 256
    n_pad = ((n + bn - 1) // bn) * bn
    xp = x if n_pad == n else jnp.pad(x, ((0, n_pad - n), (0, 0)))
    tab = _make_node_table(xp, wp, b8, bn)
    tabt = tab.reshape(n_pad, _NS, _SW).transpose(1, 0, 2).reshape(
        _NS, n_pad * _SW)

    src = edge_index[0].astype(jnp.int32)
    dst = edge_index[1].astype(jnp.int32)
    chunk = _NC * _CE
    e_pad = ((e + chunk - 1) // chunk) * chunk
    if e_pad != e:
        src = jnp.pad(src, (0, e_pad - e))
        dst = jnp.pad(dst, (0, e_pad - e))
    nch = e_pad // chunk
    s3 = src.reshape(_NC, nch, _CE)
    d3 = dst.reshape(_NC, nch, _CE)
    idx_cat = jnp.concatenate([s3, d3], axis=2).reshape(2 * e_pad)

    res = _make_edge_kernel(e_pad, n_pad)(tabt, idx_cat)
    if e_pad != e:
        res = res[:e]
    return (edge_index, res, (n, n))


# speedup vs baseline: 4.1915x; 1.4233x over previous
import functools

import jax
import jax.numpy as jnp
from jax import lax
from jax.experimental import pallas as pl
from jax.experimental.pallas import tpu as pltpu
from jax.experimental.pallas import tpu_sc as plsc

_C = 1.0
_NC = 2
_NS = 16
_L = 16
_SW = 8
_CE = 4096
_STRIP = _CE // _NS


def _node_table_body(x_ref, wp_ref, b_ref, tab_ref):
    x = x_ref[...]
    x0 = x[:, 0:1]
    total = jnp.sum(x * x, axis=1, keepdims=True)
    nsq = jnp.maximum(total - x0 * x0, 0.0)
    norm = jnp.maximum(jnp.sqrt(nsq), 1e-8)
    x0c = jnp.maximum(x0, 1.0 + 1e-6)
    dist = jnp.log(x0c + jnp.sqrt((x0c - 1.0) * (x0c + 1.0)))
    s = dist / norm
    mu = jnp.dot(x, wp_ref[...], preferred_element_type=jnp.float32) * s
    mu = mu + b_ref[0:1, :]
    mn = jnp.maximum(jnp.sqrt(jnp.sum(mu * mu, axis=1, keepdims=True)), 1e-8)
    e = jnp.exp(mn)
    ei = 1.0 / e
    ch = 0.5 * (e + ei)
    sh = 0.5 * (e - ei)
    tail = (sh / mn) * mu
    used = tail[:, : x.shape[1] - 1]
    tab_ref[...] = jnp.concatenate([ch, used], axis=1)


def _make_node_table(x, wp, b8, bn):
    n, d_in = x.shape
    grid = n // bn
    return pl.pallas_call(
        _node_table_body,
        grid=(grid,),
        in_specs=[
            pl.BlockSpec((bn, d_in), lambda i: (i, 0)),
            pl.BlockSpec((d_in, d_in), lambda i: (0, 0)),
            pl.BlockSpec((8, d_in), lambda i: (0, 0)),
        ],
        out_specs=pl.BlockSpec((bn, d_in), lambda i: (i, 0)),
        out_shape=jax.ShapeDtypeStruct((n, d_in), jnp.float32),
    )(x, wp, b8)


def _make_edge_kernel(e_total, n_rows):
    mesh = plsc.VectorSubcoreMesh(
        core_axis_name="c", subcore_axis_name="s", num_cores=_NC
    )
    epc = e_total // _NC
    nch = epc // _CE
    groups = _CE // _L
    slab_words = n_rows * _SW

    @functools.partial(
        pl.kernel,
        mesh=mesh,
        compiler_params=pltpu.CompilerParams(needs_layout_passes=False),
        out_type=jax.ShapeDtypeStruct((e_total,), jnp.float32),
        scratch_types=[
            pltpu.VMEM((slab_words,), jnp.float32),
            pltpu.VMEM((2 * _CE,), jnp.int32),
            pltpu.VMEM((2 * _CE,), jnp.int32),
            pltpu.VMEM((_CE,), jnp.float32),
            pltpu.VMEM((_NS, _STRIP), jnp.float32),
            pltpu.VMEM((_STRIP,), jnp.float32),
            pltpu.VMEM_SHARED((_NS, _CE), jnp.float32),
            pltpu.VMEM_SHARED((_NS, _CE), jnp.float32),
            pltpu.SemaphoreType.DMA,
            pltpu.SemaphoreType.DMA,
        ],
    )
    def edge_kernel(tabt_hbm, idx_hbm, out_hbm,
                    slab_v, idx0, idx1, part_v, sum_v, res_v,
                    xch0, xch1, sem_i0, sem_i1):
        cid = lax.axis_index("c")
        sid = lax.axis_index("s")
        iota = lax.iota(jnp.int32, _L)
        sgn0 = jnp.where(sid == 0, -1.0, 1.0)
        sgn = jnp.zeros((_L,), jnp.float32) + sgn0

        pltpu.sync_copy(tabt_hbm.at[sid], slab_v)

        idx_base = cid * nch * 2 * _CE
        out_base = cid * nch * _CE

        def idx_src(c):
            return idx_hbm.at[pl.ds(idx_base + c * 2 * _CE, 2 * _CE)]

        ibufs = (idx0, idx1)
        isems = (sem_i0, sem_i1)
        xbufs = (xch0, xch1)

        pltpu.sync_copy(idx_src(0), idx0)

        @pl.when(nch > 1)
        def _():
            pltpu.async_copy(idx_src(1), idx1, sem_i1)

        def process_chunk(c, ibuf, isem, ibuf_pf, isem_pf, xch):
            @pl.when(c > 0)
            def _():
                pltpu.make_async_copy(idx_src(c), ibuf, isem).wait()

            @plsc.parallel_loop(0, _CE, _L, unroll=4)
            def group_body(goff):
                sidx = ibuf[pl.ds(goff, _L)]
                didx = ibuf[pl.ds(_CE + goff, _L)]
                rs = sidx * _SW
                rd = didx * _SW
                av = [plsc.load_gather(slab_v, [rs + d] if d else [rs])
                      for d in range(_SW)]
                bv = [plsc.load_gather(slab_v, [rd + d] if d else [rd])
                      for d in range(_SW)]
                prods = [(av[0] * sgn) * bv[0]]
                prods += [av[d] * bv[d] for d in range(1, _SW)]
                while len(prods) > 1:
                    prods = [prods[i] + prods[i + 1]
                             for i in range(0, len(prods) - 1, 2)] + (
                                 [prods[-1]] if len(prods) % 2 else [])
                part_v[pl.ds(goff, _L)] = prods[0]

            pltpu.sync_copy(part_v, xch.at[sid])
            plsc.subcore_barrier()
            pltpu.sync_copy(xch.at[:, pl.ds(sid * _STRIP, _STRIP)], sum_v)

            for v in range(_STRIP // _L):
                tot = sum_v[0, pl.ds(v * _L, _L)]
                for r in range(1, _NS):
                    tot = tot + sum_v[r, pl.ds(v * _L, _L)]
                t = -_C - tot
                rr = jnp.minimum(jnp.maximum(t, 1e-10), 1.0)
                res_v[pl.ds(v * _L, _L)] = jnp.exp(-rr)
            pltpu.sync_copy(
                res_v,
                out_hbm.at[pl.ds(out_base + c * _CE + sid * _STRIP, _STRIP)])

            @pl.when(c < nch - 2)
            def _():
                pltpu.async_copy(idx_src(c + 2), ibuf, isem)

        def chunk_body(c, carry):
            @pl.when(lax.rem(c, 2) == 0)
            def _even():
                process_chunk(c, idx0, sem_i0, idx1, sem_i1, xch0)

            @pl.when(lax.rem(c, 2) == 1)
            def _odd():
                process_chunk(c, idx1, sem_i1, idx0, sem_i0, xch1)

            return carry

        lax.fori_loop(0, nch, chunk_body, 0)

    return edge_kernel


def kernel(x, edge_index, W, b):
    n, d_in = x.shape
    e = edge_index.shape[1]
    x = x.astype(jnp.float32)
    wp = jnp.concatenate(
        [jnp.zeros((1, d_in), jnp.float32), W.astype(jnp.float32)], axis=0
    )
    b8 = jnp.broadcast_to(b.astype(jnp.float32), (8, d_in))

    bn = 256
    n_pad = ((n + bn - 1) // bn) * bn
    xp = x if n_pad == n else jnp.pad(x, ((0, n_pad - n), (0, 0)))
    tab = _make_node_table(xp, wp, b8, bn)
    tabt = tab.reshape(n_pad, _NS, _SW).transpose(1, 0, 2).reshape(
        _NS, n_pad * _SW)

    src = edge_index[0].astype(jnp.int32)
    dst = edge_index[1].astype(jnp.int32)
    chunk = _NC * _CE
    e_pad = ((e + chunk - 1) // chunk) * chunk
    if e_pad != e:
        src = jnp.pad(src, (0, e_pad - e))
        dst = jnp.pad(dst, (0, e_pad - e))
    nch = e_pad // chunk
    s3 = src.reshape(_NC, nch, _CE)
    d3 = dst.reshape(_NC, nch, _CE)
    idx_cat = jnp.concatenate([s3, d3], axis=2).reshape(2 * e_pad)

    res = _make_edge_kernel(e_pad, n_pad)(tabt, idx_cat)
    if e_pad != e:
        res = res[:e]
    return (edge_index, res, (n, n))


# speedup vs baseline: 4.2521x; 1.0145x over previous
import functools

import jax
import jax.numpy as jnp
from jax import lax
from jax.experimental import pallas as pl
from jax.experimental.pallas import tpu as pltpu
from jax.experimental.pallas import tpu_sc as plsc

_C = 1.0
_NC = 2
_NS = 16
_L = 16
_SW = 8
_CE = 4096
_STRIP = _CE // _NS


def _node_table_body(x_ref, wp_ref, b_ref, tab_ref):
    x = x_ref[...]
    x0 = x[:, 0:1]
    total = jnp.sum(x * x, axis=1, keepdims=True)
    nsq = jnp.maximum(total - x0 * x0, 0.0)
    norm = jnp.maximum(jnp.sqrt(nsq), 1e-8)
    x0c = jnp.maximum(x0, 1.0 + 1e-6)
    dist = jnp.log(x0c + jnp.sqrt((x0c - 1.0) * (x0c + 1.0)))
    s = dist / norm
    mu = jnp.dot(x, wp_ref[...], preferred_element_type=jnp.float32) * s
    mu = mu + b_ref[0:1, :]
    mn = jnp.maximum(jnp.sqrt(jnp.sum(mu * mu, axis=1, keepdims=True)), 1e-8)
    e = jnp.exp(mn)
    ei = 1.0 / e
    ch = 0.5 * (e + ei)
    sh = 0.5 * (e - ei)
    tail = (sh / mn) * mu
    used = tail[:, : x.shape[1] - 1]
    tab_ref[...] = jnp.concatenate([ch, used], axis=1)


def _make_node_table(x, wp, b8, bn):
    n, d_in = x.shape
    grid = n // bn
    return pl.pallas_call(
        _node_table_body,
        grid=(grid,),
        in_specs=[
            pl.BlockSpec((bn, d_in), lambda i: (i, 0)),
            pl.BlockSpec((d_in, d_in), lambda i: (0, 0)),
            pl.BlockSpec((8, d_in), lambda i: (0, 0)),
        ],
        out_specs=pl.BlockSpec((bn, d_in), lambda i: (i, 0)),
        out_shape=jax.ShapeDtypeStruct((n, d_in), jnp.float32),
    )(x, wp, b8)


def _make_edge_kernel(e_total, n_rows):
    mesh = plsc.VectorSubcoreMesh(
        core_axis_name="c", subcore_axis_name="s", num_cores=_NC
    )
    epc = e_total // _NC
    nch = epc // _CE
    groups = _CE // _L
    slab_words = n_rows * _SW

    @functools.partial(
        pl.kernel,
        mesh=mesh,
        compiler_params=pltpu.CompilerParams(needs_layout_passes=False),
        out_type=jax.ShapeDtypeStruct((e_total,), jnp.float32),
        scratch_types=[
            pltpu.VMEM((slab_words,), jnp.float32),
            pltpu.VMEM((2 * _CE,), jnp.int32),
            pltpu.VMEM((2 * _CE,), jnp.int32),
            pltpu.VMEM((_CE,), jnp.float32),
            pltpu.VMEM((_NS, _STRIP), jnp.float32),
            pltpu.VMEM((_STRIP,), jnp.float32),
            pltpu.VMEM_SHARED((_NS, _CE), jnp.float32),
            pltpu.VMEM_SHARED((_NS, _CE), jnp.float32),
            pltpu.SemaphoreType.DMA,
            pltpu.SemaphoreType.DMA,
        ],
    )
    def edge_kernel(tabt_hbm, src_hbm, dst_hbm, out_hbm,
                    slab_v, idx0, idx1, part_v, sum_v, res_v,
                    xch0, xch1, sem_i0, sem_i1):
        cid = lax.axis_index("c")
        sid = lax.axis_index("s")
        iota = lax.iota(jnp.int32, _L)
        sgn0 = jnp.where(sid == 0, -1.0, 1.0)
        sgn = jnp.zeros((_L,), jnp.float32) + sgn0

        pltpu.sync_copy(tabt_hbm.at[sid], slab_v)

        edge_base = cid * nch * _CE
        out_base = cid * nch * _CE

        def idx_descs(c, ibuf, isem):
            off = edge_base + c * _CE
            return (
                pltpu.make_async_copy(src_hbm.at[pl.ds(off, _CE)],
                                      ibuf.at[pl.ds(0, _CE)], isem),
                pltpu.make_async_copy(dst_hbm.at[pl.ds(off, _CE)],
                                      ibuf.at[pl.ds(_CE, _CE)], isem),
            )

        def idx_start(c, ibuf, isem):
            d1, d2 = idx_descs(c, ibuf, isem)
            d1.start()
            d2.start()

        def idx_wait(c, ibuf, isem):
            d1, d2 = idx_descs(c, ibuf, isem)
            d1.wait()
            d2.wait()

        idx_start(0, idx0, sem_i0)
        idx_wait(0, idx0, sem_i0)

        @pl.when(nch > 1)
        def _():
            idx_start(1, idx1, sem_i1)

        def process_chunk(c, ibuf, isem, xch):
            @pl.when(c > 0)
            def _():
                idx_wait(c, ibuf, isem)

            @plsc.parallel_loop(0, _CE, _L, unroll=4)
            def group_body(goff):
                sidx = ibuf[pl.ds(goff, _L)]
                didx = ibuf[pl.ds(_CE + goff, _L)]
                rs = sidx * _SW
                rd = didx * _SW
                av = [plsc.load_gather(slab_v, [rs + d] if d else [rs])
                      for d in range(_SW)]
                bv = [plsc.load_gather(slab_v, [rd + d] if d else [rd])
                      for d in range(_SW)]
                prods = [(av[0] * sgn) * bv[0]]
                prods += [av[d] * bv[d] for d in range(1, _SW)]
                while len(prods) > 1:
                    prods = [prods[i] + prods[i + 1]
                             for i in range(0, len(prods) - 1, 2)] + (
                                 [prods[-1]] if len(prods) % 2 else [])
                part_v[pl.ds(goff, _L)] = prods[0]

            pltpu.sync_copy(part_v, xch.at[sid])
            plsc.subcore_barrier()
            pltpu.sync_copy(xch.at[:, pl.ds(sid * _STRIP, _STRIP)], sum_v)

            for v in range(_STRIP // _L):
                tot = sum_v[0, pl.ds(v * _L, _L)]
                for r in range(1, _NS):
                    tot = tot + sum_v[r, pl.ds(v * _L, _L)]
                t = -_C - tot
                rr = jnp.minimum(jnp.maximum(t, 1e-10), 1.0)
                res_v[pl.ds(v * _L, _L)] = jnp.exp(-rr)
            pltpu.sync_copy(
                res_v,
                out_hbm.at[pl.ds(out_base + c * _CE + sid * _STRIP, _STRIP)])

            @pl.when(c < nch - 2)
            def _():
                idx_start(c + 2, ibuf, isem)

        def chunk_body(c, carry):
            @pl.when(lax.rem(c, 2) == 0)
            def _even():
                process_chunk(c, idx0, sem_i0, xch0)

            @pl.when(lax.rem(c, 2) == 1)
            def _odd():
                process_chunk(c, idx1, sem_i1, xch1)

            return carry

        lax.fori_loop(0, nch, chunk_body, 0)

    return edge_kernel


def kernel(x, edge_index, W, b):
    n, d_in = x.shape
    e = edge_index.shape[1]
    x = x.astype(jnp.float32)
    wp = jnp.concatenate(
        [jnp.zeros((1, d_in), jnp.float32), W.astype(jnp.float32)], axis=0
    )
    b8 = jnp.broadcast_to(b.astype(jnp.float32), (8, d_in))

    bn = 256
    n_pad = ((n + bn - 1) // bn) * bn
    xp = x if n_pad == n else jnp.pad(x, ((0, n_pad - n), (0, 0)))
    tab = _make_node_table(xp, wp, b8, bn)
    tabt = tab.reshape(n_pad, _NS, _SW).transpose(1, 0, 2).reshape(
        _NS, n_pad * _SW)

    src = edge_index[0].astype(jnp.int32)
    dst = edge_index[1].astype(jnp.int32)
    chunk = _NC * _CE
    e_pad = ((e + chunk - 1) // chunk) * chunk
    if e_pad != e:
        src = jnp.pad(src, (0, e_pad - e))
        dst = jnp.pad(dst, (0, e_pad - e))

    res = _make_edge_kernel(e_pad, n_pad)(tabt, src, dst)
    if e_pad != e:
        res = res[:e]
    return (edge_index, res, (n, n))
